# Initial kernel scaffold; baseline (speedup 1.0000x reference)
import jax
import jax.numpy as jnp
from jax.experimental import pallas as pl


def kernel(gnn_in, centers, Wf1, bf1, Ws1, bs1, g1, be1, Wf2, bf2, Ws2, bs2, g2, be2, edge_index):
    raise NotImplementedError("write your pallas kernel here")


# speedup vs baseline: 20.2241x; 20.2241x over previous
import numpy as np
import jax
import jax.numpy as jnp
from jax.experimental import pallas as pl
from jax.experimental.pallas import tpu as pltpu

_F = 64
_N_SAMPLES = 142
_sizes = np.arange(_N_SAMPLES)
_offs = np.concatenate([[0], np.cumsum(_sizes)]).astype(np.int64)
_N = int(_offs[-1])
_NPAD = 10024
_JW = 160
_JH = _JW // 2

_tiles = []
for _k in range(_N_SAMPLES):
    _n = int(_sizes[_k])
    _o = int(_offs[_k])
    if _n < 1:
        continue
    _ob = (_o // 8) * 8
    _dj = _o - _ob
    _nt = -(-(_dj + _n) // 8)
    for _a in range(_nt):
        _ib = _ob + 8 * _a
        _tiles.append((_ib, _ob, _dj, _o - _ib, _n))
_T = len(_tiles)
_TIB = np.array([t[0] for t in _tiles], dtype=np.int32)
_TOB = np.array([t[1] for t in _tiles], dtype=np.int32)
_TDJ = np.array([t[2] for t in _tiles], dtype=np.int32)
_TDI = np.array([t[3] for t in _tiles], dtype=np.int32)
_TN = np.array([t[4] for t in _tiles], dtype=np.int32)


def _sigmoid(x):
    return 1.0 / (1.0 + jnp.exp(-x))


def _softplus(x):
    return jnp.maximum(x, 0.0) + jnp.log(1.0 + jnp.exp(-jnp.abs(x)))


def _body(x_ref, c_ref, wc1_ref, we1_ref, b1_ref, gb1_ref,
          wc2_ref, we2_ref, b2_ref, gb2_ref,
          tib_ref, tob_ref, tdj_ref, tdi_ref, tn_ref,
          out_ref, pq_ref, agg_ref, x2_ref):

    def project(xv, wc_ref, we_ref, b_ref):
        cc = jnp.dot(c_ref[...], we_ref[...],
                     preferred_element_type=jnp.float32)
        cf = cc[:, 0:64]
        cs = cc[:, 64:128]
        adj = jnp.concatenate([cf, -cf, cs, -cs], axis=1) + b_ref[...]
        pq_ref[...] = jnp.dot(xv, wc_ref[...],
                              preferred_element_type=jnp.float32) + adj

    def pair_loop():
        def tb(t, carry):
            ib = pl.multiple_of(tib_ref[t], 8)
            ob = pl.multiple_of(tob_ref[t], 8)
            dj = tdj_ref[t]
            di = tdi_ref[t]
            n = tn_ref[t]
            pf = pq_ref[pl.ds(ib, 8), 0:64]
            qf_d = pq_ref[pl.ds(ib, 8), 64:128]
            ps = pq_ref[pl.ds(ib, 8), 128:192]
            qs_d = pq_ref[pl.ds(ib, 8), 192:256]
            jf = jnp.concatenate(
                [pq_ref[pl.ds(ob, _JH), 64:128],
                 pq_ref[pl.ds(ob + _JH, _JH), 64:128]], axis=1)
            js = jnp.concatenate(
                [pq_ref[pl.ds(ob, _JH), 192:256],
                 pq_ref[pl.ds(ob + _JH, _JH), 192:256]], axis=1)
            pfd = jnp.concatenate([pf, pf], axis=1)[:, None, :]
            psd = jnp.concatenate([ps, ps], axis=1)[:, None, :]
            a = pfd + jf[None]
            b = psd + js[None]
            m = _sigmoid(a) * _softplus(b)
            r = jax.lax.broadcasted_iota(jnp.int32, (_JH, 128), 0)
            l = jax.lax.broadcasted_iota(jnp.int32, (_JH, 128), 1)
            jloc = r + jnp.where(l >= 64, _JH, 0)
            jok = (jloc >= dj) & (jloc < dj + n)
            m = jnp.where(jok[None], m, 0.0)
            s = jnp.sum(m, axis=1)
            aggt = s[:, 0:64] + s[:, 64:128]
            diag = _sigmoid(pf + qf_d) * _softplus(ps + qs_d)
            rows8 = jax.lax.broadcasted_iota(jnp.int32, (8, 1), 0)
            iok = (rows8 >= di) & (rows8 < di + n)
            old = agg_ref[pl.ds(ib, 8), :]
            agg_ref[pl.ds(ib, 8), :] = jnp.where(iok, aggt - diag, old)
            return carry
        jax.lax.fori_loop(0, _T, tb, 0)

    def bn_res_relu(xv, gb_ref):
        agg = agg_ref[...]
        rows = jax.lax.broadcasted_iota(jnp.int32, (_NPAD, 1), 0)
        valid = rows < _N
        aggm = jnp.where(valid, agg, 0.0)
        mu = jnp.sum(aggm, axis=0, keepdims=True) * (1.0 / _N)
        d = jnp.where(valid, aggm - mu, 0.0)
        var = jnp.sum(d * d, axis=0, keepdims=True) * (1.0 / _N)
        g = gb_ref[0:1, :]
        be = gb_ref[1:2, :]
        scale = g * jax.lax.rsqrt(var + 1e-5)
        return jnp.maximum(xv + (agg - mu) * scale + be, 0.0)

    x1 = x_ref[...]
    project(x1, wc1_ref, we1_ref, b1_ref)
    pair_loop()
    x2 = bn_res_relu(x1, gb1_ref)
    x2_ref[...] = x2
    project(x2_ref[...], wc2_ref, we2_ref, b2_ref)
    pair_loop()
    out_ref[...] = bn_res_relu(x2_ref[...], gb2_ref)


def kernel(gnn_in, centers, Wf1, bf1, Ws1, bs1, g1, be1,
           Wf2, bf2, Ws2, bs2, g2, be2, edge_index):
    f32 = jnp.float32
    xpad = jnp.zeros((_NPAD, _F), f32).at[:_N].set(gnn_in)
    cpad = jnp.zeros((_NPAD, 2), f32).at[:_N].set(centers)

    def pack(Wf, bf, Ws, bs, g, be):
        wc = jnp.concatenate([Wf[0:64], Wf[64:128], Ws[0:64], Ws[64:128]],
                             axis=1)
        we = jnp.concatenate([Wf[128:130], Ws[128:130]], axis=1)
        z = jnp.zeros((_F,), f32)
        b = jnp.concatenate([bf, z, bs, z])[None, :]
        gb = jnp.stack([g, be])
        return wc, we, b, gb

    wc1, we1, b1, gb1 = pack(Wf1, bf1, Ws1, bs1, g1, be1)
    wc2, we2, b2, gb2 = pack(Wf2, bf2, Ws2, bs2, g2, be2)

    vmem = pl.BlockSpec(memory_space=pltpu.VMEM)
    sspec = pl.BlockSpec(memory_space=pltpu.SMEM)

    out = pl.pallas_call(
        _body,
        out_shape=jax.ShapeDtypeStruct((_NPAD, _F), f32),
        in_specs=[vmem] * 10 + [sspec] * 5,
        out_specs=vmem,
        scratch_shapes=[
            pltpu.VMEM((_NPAD, 4 * _F), f32),
            pltpu.VMEM((_NPAD, _F), f32),
            pltpu.VMEM((_NPAD, _F), f32),
        ],
    )(xpad, cpad, wc1, we1, b1, gb1, wc2, we2, b2, gb2,
      jnp.asarray(_TIB), jnp.asarray(_TOB), jnp.asarray(_TDJ),
      jnp.asarray(_TDI), jnp.asarray(_TN))
    return out[:_N]


# speedup vs baseline: 26.2961x; 1.3002x over previous
import numpy as np
import jax
import jax.numpy as jnp
from jax.experimental import pallas as pl
from jax.experimental.pallas import tpu as pltpu

_F = 64
_N_SAMPLES = 142
_sizes = np.arange(_N_SAMPLES)
_offs = np.concatenate([[0], np.cumsum(_sizes)]).astype(np.int64)
_N = int(_offs[-1])
_NPAD = 10024
_JW = 160
_JH = _JW // 2

_tiles_by_class = {}
for _k in range(_N_SAMPLES):
    _n = int(_sizes[_k])
    _o = int(_offs[_k])
    if _n < 1:
        continue
    _ob = (_o // 8) * 8
    _dj = _o - _ob
    _w = -(-(_dj + _n) // 16) * 16
    _nt = -(-(_dj + _n) // 8)
    for _a in range(_nt):
        _ib = _ob + 8 * _a
        _tiles_by_class.setdefault(_w, []).append(
            (_ib, _ob, _dj, _o - _ib, _n))
_CLASSES = []
_tiles = []
for _w in sorted(_tiles_by_class):
    _s = len(_tiles)
    _tiles.extend(_tiles_by_class[_w])
    _CLASSES.append((_w, _s, len(_tiles)))
_T = len(_tiles)
_TIB = np.array([t[0] for t in _tiles], dtype=np.int32)
_TOB = np.array([t[1] for t in _tiles], dtype=np.int32)
_TDJ = np.array([t[2] for t in _tiles], dtype=np.int32)
_TDI = np.array([t[3] for t in _tiles], dtype=np.int32)
_TN = np.array([t[4] for t in _tiles], dtype=np.int32)


_LOG2E = float(np.log2(np.e))
_LN2 = float(np.log(2.0))


def _sigmoid(x):
    return 1.0 / (1.0 + jnp.exp2(x * (-_LOG2E)))


def _softplus2(x):
    return jnp.log2(1.0 + jnp.exp2(jnp.minimum(x, 80.0) * _LOG2E))


def _body(x_ref, c_ref, wc1_ref, we1_ref, b1_ref, gb1_ref,
          wc2_ref, we2_ref, b2_ref, gb2_ref,
          tib_ref, tob_ref, tdj_ref, tdi_ref, tn_ref,
          out_ref, pq_ref, agg_ref, x2_ref):

    def project(xv, wc_ref, we_ref, b_ref):
        cc = jnp.dot(c_ref[...], we_ref[...],
                     preferred_element_type=jnp.float32)
        cf = cc[:, 0:64]
        cs = cc[:, 64:128]
        adj = jnp.concatenate([cf, -cf, cs, -cs], axis=1) + b_ref[...]
        pq_ref[...] = jnp.dot(xv, wc_ref[...],
                              preferred_element_type=jnp.float32) + adj

    def pair_loop():
        def make_tb(jh):
            def tb(t, carry):
                ib = pl.multiple_of(tib_ref[t], 8)
                ob = pl.multiple_of(tob_ref[t], 8)
                dj = tdj_ref[t]
                di = tdi_ref[t]
                n = tn_ref[t]
                pf = pq_ref[pl.ds(ib, 8), 0:64]
                qf_d = pq_ref[pl.ds(ib, 8), 64:128]
                ps = pq_ref[pl.ds(ib, 8), 128:192]
                qs_d = pq_ref[pl.ds(ib, 8), 192:256]
                jf = jnp.concatenate(
                    [pq_ref[pl.ds(ob, jh), 64:128],
                     pq_ref[pl.ds(ob + jh, jh), 64:128]], axis=1)
                js = jnp.concatenate(
                    [pq_ref[pl.ds(ob, jh), 192:256],
                     pq_ref[pl.ds(ob + jh, jh), 192:256]], axis=1)
                pfd = jnp.concatenate([pf, pf], axis=1)[:, None, :]
                psd = jnp.concatenate([ps, ps], axis=1)[:, None, :]
                a = pfd + jf[None]
                b = psd + js[None]
                m = _sigmoid(a) * _softplus2(b)
                r = jax.lax.broadcasted_iota(jnp.int32, (jh, 128), 0)
                l = jax.lax.broadcasted_iota(jnp.int32, (jh, 128), 1)
                jloc = r + jnp.where(l >= 64, jh, 0)
                jok = (jloc >= dj) & (jloc < dj + n)
                m = jnp.where(jok[None], m, 0.0)
                s = jnp.sum(m, axis=1)
                diag = _sigmoid(pf + qf_d) * _softplus2(ps + qs_d)
                aggt = (s[:, 0:64] + s[:, 64:128] - diag) * _LN2
                rows8 = jax.lax.broadcasted_iota(jnp.int32, (8, 1), 0)
                iok = (rows8 >= di) & (rows8 < di + n)
                old = agg_ref[pl.ds(ib, 8), :]
                agg_ref[pl.ds(ib, 8), :] = jnp.where(iok, aggt, old)
                return carry
            return tb
        for w, s, e in _CLASSES:
            jax.lax.fori_loop(s, e, make_tb(w // 2), 0)

    def bn_res_relu(xv, gb_ref):
        agg = agg_ref[...]
        rows = jax.lax.broadcasted_iota(jnp.int32, (_NPAD, 1), 0)
        valid = rows < _N
        aggm = jnp.where(valid, agg, 0.0)
        mu = jnp.sum(aggm, axis=0, keepdims=True) * (1.0 / _N)
        d = jnp.where(valid, aggm - mu, 0.0)
        var = jnp.sum(d * d, axis=0, keepdims=True) * (1.0 / _N)
        g = gb_ref[0:1, :]
        be = gb_ref[1:2, :]
        scale = g * jax.lax.rsqrt(var + 1e-5)
        return jnp.maximum(xv + (agg - mu) * scale + be, 0.0)

    x1 = x_ref[...]
    project(x1, wc1_ref, we1_ref, b1_ref)
    pair_loop()
    x2 = bn_res_relu(x1, gb1_ref)
    x2_ref[...] = x2
    project(x2_ref[...], wc2_ref, we2_ref, b2_ref)
    pair_loop()
    out_ref[...] = bn_res_relu(x2_ref[...], gb2_ref)


def kernel(gnn_in, centers, Wf1, bf1, Ws1, bs1, g1, be1,
           Wf2, bf2, Ws2, bs2, g2, be2, edge_index):
    f32 = jnp.float32
    xpad = jnp.zeros((_NPAD, _F), f32).at[:_N].set(gnn_in)
    cpad = jnp.zeros((_NPAD, 2), f32).at[:_N].set(centers)

    def pack(Wf, bf, Ws, bs, g, be):
        wc = jnp.concatenate([Wf[0:64], Wf[64:128], Ws[0:64], Ws[64:128]],
                             axis=1)
        we = jnp.concatenate([Wf[128:130], Ws[128:130]], axis=1)
        z = jnp.zeros((_F,), f32)
        b = jnp.concatenate([bf, z, bs, z])[None, :]
        gb = jnp.stack([g, be])
        return wc, we, b, gb

    wc1, we1, b1, gb1 = pack(Wf1, bf1, Ws1, bs1, g1, be1)
    wc2, we2, b2, gb2 = pack(Wf2, bf2, Ws2, bs2, g2, be2)

    vmem = pl.BlockSpec(memory_space=pltpu.VMEM)
    sspec = pl.BlockSpec(memory_space=pltpu.SMEM)

    out = pl.pallas_call(
        _body,
        out_shape=jax.ShapeDtypeStruct((_NPAD, _F), f32),
        in_specs=[vmem] * 10 + [sspec] * 5,
        out_specs=vmem,
        scratch_shapes=[
            pltpu.VMEM((_NPAD, 4 * _F), f32),
            pltpu.VMEM((_NPAD, _F), f32),
            pltpu.VMEM((_NPAD, _F), f32),
        ],
    )(xpad, cpad, wc1, we1, b1, gb1, wc2, we2, b2, gb2,
      jnp.asarray(_TIB), jnp.asarray(_TOB), jnp.asarray(_TDJ),
      jnp.asarray(_TDI), jnp.asarray(_TN))
    return out[:_N]


# speedup vs baseline: 26.5017x; 1.0078x over previous
import numpy as np
import jax
import jax.numpy as jnp
from jax.experimental import pallas as pl
from jax.experimental.pallas import tpu as pltpu

_F = 64
_N_SAMPLES = 142
_sizes = np.arange(_N_SAMPLES)
_offs = np.concatenate([[0], np.cumsum(_sizes)]).astype(np.int64)
_N = int(_offs[-1])
_NPAD = 10024
_JW = 160
_JH = _JW // 2

_tiles_by_class = {}
for _k in range(_N_SAMPLES):
    _n = int(_sizes[_k])
    _o = int(_offs[_k])
    if _n < 1:
        continue
    _ob = (_o // 8) * 8
    _dj = _o - _ob
    _w = -(-(_dj + _n) // 16) * 16
    _nt = -(-(_dj + _n) // 8)
    for _a in range(_nt):
        _ib = _ob + 8 * _a
        _tiles_by_class.setdefault(_w, []).append(
            (_ib, _ob, _dj, _o - _ib, _n))
_CLASSES = []
_tiles = []
for _w in sorted(_tiles_by_class):
    _s = len(_tiles)
    _tiles.extend(_tiles_by_class[_w])
    _CLASSES.append((_w, _s, len(_tiles)))
_T = len(_tiles)
_TIB = np.array([t[0] for t in _tiles], dtype=np.int32)
_TOB = np.array([t[1] for t in _tiles], dtype=np.int32)
_TDJ = np.array([t[2] for t in _tiles], dtype=np.int32)
_TDI = np.array([t[3] for t in _tiles], dtype=np.int32)
_TN = np.array([t[4] for t in _tiles], dtype=np.int32)


_LOG2E = float(np.log2(np.e))
_LN2 = float(np.log(2.0))
_SPCLAMP = 80.0 * _LOG2E


def _sig2(a2):
    return 1.0 / (1.0 + jnp.exp2(a2))


def _sp2(b2):
    return jnp.log2(1.0 + jnp.exp2(jnp.minimum(b2, _SPCLAMP)))


def _body(x_ref, c_ref, wc1_ref, we1_ref, b1_ref, gb1_ref,
          wc2_ref, we2_ref, b2_ref, gb2_ref,
          tib_ref, tob_ref, tdj_ref, tdi_ref, tn_ref,
          out_ref, pq_ref, agg_ref, x2_ref):

    def project(xv, wc_ref, we_ref, b_ref):
        cc = jnp.dot(c_ref[...], we_ref[...],
                     preferred_element_type=jnp.float32)
        cf = cc[:, 0:64]
        cs = cc[:, 64:128]
        adj = jnp.concatenate([cf, -cf, cs, -cs], axis=1) + b_ref[...]
        pq_ref[...] = jnp.dot(xv, wc_ref[...],
                              preferred_element_type=jnp.float32) + adj

    def pair_loop():
        def make_tb(jh):
            def tb(t, carry):
                ib = pl.multiple_of(tib_ref[t], 8)
                ob = pl.multiple_of(tob_ref[t], 8)
                dj = tdj_ref[t]
                di = tdi_ref[t]
                n = tn_ref[t]
                pf = pq_ref[pl.ds(ib, 8), 0:64]
                ps = pq_ref[pl.ds(ib, 8), 128:192]
                jf = jnp.concatenate(
                    [pq_ref[pl.ds(ob, jh), 64:128],
                     pq_ref[pl.ds(ob + jh, jh), 64:128]], axis=1)
                js = jnp.concatenate(
                    [pq_ref[pl.ds(ob, jh), 192:256],
                     pq_ref[pl.ds(ob + jh, jh), 192:256]], axis=1)
                pfd = jnp.concatenate([pf, pf], axis=1)[:, None, :]
                psd = jnp.concatenate([ps, ps], axis=1)[:, None, :]
                a = pfd + jf[None]
                b = psd + js[None]
                m = _sig2(a) * _sp2(b)
                r = jax.lax.broadcasted_iota(jnp.int32, (jh, 128), 0)
                l = jax.lax.broadcasted_iota(jnp.int32, (jh, 128), 1)
                jloc = r + jnp.where(l >= 64, jh, 0)
                jok = (jloc >= dj) & (jloc < dj + n)
                m = jnp.where(jok[None], m, 0.0)
                s = jnp.sum(m, axis=1)
                aggt = (s[:, 0:64] + s[:, 64:128]) * _LN2
                rows8 = jax.lax.broadcasted_iota(jnp.int32, (8, 1), 0)
                iok = (rows8 >= di) & (rows8 < di + n)
                old = agg_ref[pl.ds(ib, 8), :]
                agg_ref[pl.ds(ib, 8), :] = jnp.where(iok, aggt, old)
                return carry
            return tb
        for w, s, e in _CLASSES:
            jax.lax.fori_loop(s, e, make_tb(w // 2), 0)

    def bn_res_relu(xv, gb_ref):
        pq = pq_ref[...]
        diag = _sig2(pq[:, 0:64] + pq[:, 64:128]) * \
            _sp2(pq[:, 128:192] + pq[:, 192:256]) * _LN2
        agg = agg_ref[...] - diag
        rows = jax.lax.broadcasted_iota(jnp.int32, (_NPAD, 1), 0)
        valid = rows < _N
        aggm = jnp.where(valid, agg, 0.0)
        mu = jnp.sum(aggm, axis=0, keepdims=True) * (1.0 / _N)
        d = jnp.where(valid, aggm - mu, 0.0)
        var = jnp.sum(d * d, axis=0, keepdims=True) * (1.0 / _N)
        g = gb_ref[0:1, :]
        be = gb_ref[1:2, :]
        scale = g * jax.lax.rsqrt(var + 1e-5)
        return jnp.maximum(xv + (agg - mu) * scale + be, 0.0)

    x1 = x_ref[...]
    project(x1, wc1_ref, we1_ref, b1_ref)
    pair_loop()
    x2 = bn_res_relu(x1, gb1_ref)
    x2_ref[...] = x2
    project(x2_ref[...], wc2_ref, we2_ref, b2_ref)
    pair_loop()
    out_ref[...] = bn_res_relu(x2_ref[...], gb2_ref)


def kernel(gnn_in, centers, Wf1, bf1, Ws1, bs1, g1, be1,
           Wf2, bf2, Ws2, bs2, g2, be2, edge_index):
    f32 = jnp.float32
    xpad = jnp.zeros((_NPAD, _F), f32).at[:_N].set(gnn_in)
    cpad = jnp.zeros((_NPAD, 2), f32).at[:_N].set(centers)

    def pack(Wf, bf, Ws, bs, g, be):
        Wfs = Wf * (-_LOG2E)
        Wss = Ws * _LOG2E
        wc = jnp.concatenate([Wfs[0:64], Wfs[64:128], Wss[0:64],
                              Wss[64:128]], axis=1)
        we = jnp.concatenate([Wfs[128:130], Wss[128:130]], axis=1)
        z = jnp.zeros((_F,), f32)
        b = jnp.concatenate([bf * (-_LOG2E), z, bs * _LOG2E, z])[None, :]
        gb = jnp.stack([g, be])
        return wc, we, b, gb

    wc1, we1, b1, gb1 = pack(Wf1, bf1, Ws1, bs1, g1, be1)
    wc2, we2, b2, gb2 = pack(Wf2, bf2, Ws2, bs2, g2, be2)

    vmem = pl.BlockSpec(memory_space=pltpu.VMEM)
    sspec = pl.BlockSpec(memory_space=pltpu.SMEM)

    out = pl.pallas_call(
        _body,
        out_shape=jax.ShapeDtypeStruct((_NPAD, _F), f32),
        in_specs=[vmem] * 10 + [sspec] * 5,
        out_specs=vmem,
        scratch_shapes=[
            pltpu.VMEM((_NPAD, 4 * _F), f32),
            pltpu.VMEM((_NPAD, _F), f32),
            pltpu.VMEM((_NPAD, _F), f32),
        ],
    )(xpad, cpad, wc1, we1, b1, gb1, wc2, we2, b2, gb2,
      jnp.asarray(_TIB), jnp.asarray(_TOB), jnp.asarray(_TDJ),
      jnp.asarray(_TDI), jnp.asarray(_TN))
    return out[:_N]


# speedup vs baseline: 34.4416x; 1.2996x over previous
import numpy as np
import jax
import jax.numpy as jnp
from jax.experimental import pallas as pl
from jax.experimental.pallas import tpu as pltpu

_F = 64
_N_SAMPLES = 142
_sizes = np.arange(_N_SAMPLES)
_offs = np.concatenate([[0], np.cumsum(_sizes)]).astype(np.int64)
_N = int(_offs[-1])
_NPAD = 10024
_JW = 160
_JH = _JW // 2

_tiles_by_class = {}
for _k in range(_N_SAMPLES):
    _n = int(_sizes[_k])
    _o = int(_offs[_k])
    if _n < 1:
        continue
    _ob = (_o // 8) * 8
    _dj = _o - _ob
    _w = -(-(_dj + _n) // 16) * 16
    _istep = 16 if _w >= 48 else 8
    _nt = -(-(_dj + _n) // _istep)
    for _a in range(_nt):
        _ib = _ob + _istep * _a
        _tiles_by_class.setdefault((_w, _istep), []).append(
            (_ib, _ob, _dj, _o - _ib, _n))
_CLASSES = []
_tiles = []
for _wi in sorted(_tiles_by_class):
    _s = len(_tiles)
    _tiles.extend(_tiles_by_class[_wi])
    _CLASSES.append((_wi[0], _wi[1], _s, len(_tiles)))
_T = len(_tiles)
_TIB = np.array([t[0] for t in _tiles], dtype=np.int32)
_TOB = np.array([t[1] for t in _tiles], dtype=np.int32)
_TDJ = np.array([t[2] for t in _tiles], dtype=np.int32)
_TDI = np.array([t[3] for t in _tiles], dtype=np.int32)
_TN = np.array([t[4] for t in _tiles], dtype=np.int32)


_LOG2E = float(np.log2(np.e))
_LN2 = float(np.log(2.0))
_SPCLAMP = 80.0 * _LOG2E


def _sig2(a2):
    return 1.0 / (1.0 + jnp.exp2(a2))


def _sp2(b2):
    return jnp.log2(1.0 + jnp.exp2(jnp.minimum(b2, _SPCLAMP)))


def _body(x_ref, c_ref, wc1_ref, we1_ref, b1_ref, gb1_ref,
          wc2_ref, we2_ref, b2_ref, gb2_ref,
          tib_ref, tob_ref, tdj_ref, tdi_ref, tn_ref,
          out_ref, pq_ref, agg_ref, x2_ref):

    def project(xv, wc_ref, we_ref, b_ref):
        cc = jnp.dot(c_ref[...], we_ref[...],
                     preferred_element_type=jnp.float32)
        cf = cc[:, 0:64]
        cs = cc[:, 64:128]
        adj = jnp.concatenate([cf, -cf, cs, -cs], axis=1) + b_ref[...]
        pq_ref[...] = jnp.dot(xv, wc_ref[...],
                              preferred_element_type=jnp.float32) + adj

    def pair_loop():
        def make_tb(jh, ih):
            def tb(t, carry):
                ib = pl.multiple_of(tib_ref[t], 8)
                ob = pl.multiple_of(tob_ref[t], 8)
                dj = tdj_ref[t]
                di = tdi_ref[t]
                n = tn_ref[t]
                pf = pq_ref[pl.ds(ib, ih), 0:64]
                ps = pq_ref[pl.ds(ib, ih), 128:192]
                jf = jnp.concatenate(
                    [pq_ref[pl.ds(ob, jh), 64:128],
                     pq_ref[pl.ds(ob + jh, jh), 64:128]], axis=1)
                js = jnp.concatenate(
                    [pq_ref[pl.ds(ob, jh), 192:256],
                     pq_ref[pl.ds(ob + jh, jh), 192:256]], axis=1)
                pfd = jnp.concatenate([pf, pf], axis=1)[:, None, :]
                psd = jnp.concatenate([ps, ps], axis=1)[:, None, :]
                a = pfd + jf[None]
                b = psd + js[None]
                m = _sig2(a) * _sp2(b)
                r = jax.lax.broadcasted_iota(jnp.int32, (jh, 128), 0)
                l = jax.lax.broadcasted_iota(jnp.int32, (jh, 128), 1)
                jloc = r + jnp.where(l >= 64, jh, 0)
                jok = (jloc >= dj) & (jloc < dj + n)
                m = jnp.where(jok[None], m, 0.0)
                s = jnp.sum(m, axis=1)
                aggt = (s[:, 0:64] + s[:, 64:128]) * _LN2
                rowsi = jax.lax.broadcasted_iota(jnp.int32, (ih, 1), 0)
                iok = (rowsi >= di) & (rowsi < di + n)
                old = agg_ref[pl.ds(ib, ih), :]
                agg_ref[pl.ds(ib, ih), :] = jnp.where(iok, aggt, old)
                return carry
            return tb
        for w, ih, s, e in _CLASSES:
            jax.lax.fori_loop(s, e, make_tb(w // 2, ih), 0)

    def bn_res_relu(xv, gb_ref):
        pq = pq_ref[...]
        diag = _sig2(pq[:, 0:64] + pq[:, 64:128]) * \
            _sp2(pq[:, 128:192] + pq[:, 192:256]) * _LN2
        agg = agg_ref[...] - diag
        rows = jax.lax.broadcasted_iota(jnp.int32, (_NPAD, 1), 0)
        valid = rows < _N
        aggm = jnp.where(valid, agg, 0.0)
        mu = jnp.sum(aggm, axis=0, keepdims=True) * (1.0 / _N)
        d = jnp.where(valid, aggm - mu, 0.0)
        var = jnp.sum(d * d, axis=0, keepdims=True) * (1.0 / _N)
        g = gb_ref[0:1, :]
        be = gb_ref[1:2, :]
        scale = g * jax.lax.rsqrt(var + 1e-5)
        return jnp.maximum(xv + (agg - mu) * scale + be, 0.0)

    x1 = x_ref[...]
    project(x1, wc1_ref, we1_ref, b1_ref)
    pair_loop()
    x2 = bn_res_relu(x1, gb1_ref)
    x2_ref[...] = x2
    project(x2_ref[...], wc2_ref, we2_ref, b2_ref)
    pair_loop()
    out_ref[...] = bn_res_relu(x2_ref[...], gb2_ref)


def kernel(gnn_in, centers, Wf1, bf1, Ws1, bs1, g1, be1,
           Wf2, bf2, Ws2, bs2, g2, be2, edge_index):
    f32 = jnp.float32
    xpad = jnp.zeros((_NPAD, _F), f32).at[:_N].set(gnn_in)
    cpad = jnp.zeros((_NPAD, 2), f32).at[:_N].set(centers)

    def pack(Wf, bf, Ws, bs, g, be):
        Wfs = Wf * (-_LOG2E)
        Wss = Ws * _LOG2E
        wc = jnp.concatenate([Wfs[0:64], Wfs[64:128], Wss[0:64],
                              Wss[64:128]], axis=1)
        we = jnp.concatenate([Wfs[128:130], Wss[128:130]], axis=1)
        z = jnp.zeros((_F,), f32)
        b = jnp.concatenate([bf * (-_LOG2E), z, bs * _LOG2E, z])[None, :]
        gb = jnp.stack([g, be])
        return wc, we, b, gb

    wc1, we1, b1, gb1 = pack(Wf1, bf1, Ws1, bs1, g1, be1)
    wc2, we2, b2, gb2 = pack(Wf2, bf2, Ws2, bs2, g2, be2)

    vmem = pl.BlockSpec(memory_space=pltpu.VMEM)
    sspec = pl.BlockSpec(memory_space=pltpu.SMEM)

    out = pl.pallas_call(
        _body,
        out_shape=jax.ShapeDtypeStruct((_NPAD, _F), f32),
        in_specs=[vmem] * 10 + [sspec] * 5,
        out_specs=vmem,
        scratch_shapes=[
            pltpu.VMEM((_NPAD, 4 * _F), f32),
            pltpu.VMEM((_NPAD, _F), f32),
            pltpu.VMEM((_NPAD, _F), f32),
        ],
    )(xpad, cpad, wc1, we1, b1, gb1, wc2, we2, b2, gb2,
      jnp.asarray(_TIB), jnp.asarray(_TOB), jnp.asarray(_TDJ),
      jnp.asarray(_TDI), jnp.asarray(_TN))
    return out[:_N]


# speedup vs baseline: 36.7164x; 1.0660x over previous
import numpy as np
import jax
import jax.numpy as jnp
from jax.experimental import pallas as pl
from jax.experimental.pallas import tpu as pltpu

_F = 64
_N_SAMPLES = 142
_sizes = np.arange(_N_SAMPLES)
_offs = np.concatenate([[0], np.cumsum(_sizes)]).astype(np.int64)
_N = int(_offs[-1])
_NPAD = 10024
_JW = 160
_JH = _JW // 2

_tiles_by_class = {}
for _k in range(_N_SAMPLES):
    _n = int(_sizes[_k])
    _o = int(_offs[_k])
    if _n < 1:
        continue
    _ob = (_o // 8) * 8
    _dj = _o - _ob
    _w = -(-(_dj + _n) // 16) * 16
    _istep = 32 if _w >= 96 else (16 if _w >= 48 else 8)
    _nt = -(-(_dj + _n) // _istep)
    for _a in range(_nt):
        _ib = _ob + _istep * _a
        _tiles_by_class.setdefault((_w, _istep), []).append(
            (_ib, _ob, _dj, _o - _ib, _n))
_CLASSES = []
_tiles = []
for _wi in sorted(_tiles_by_class):
    _s = len(_tiles)
    _tiles.extend(_tiles_by_class[_wi])
    _CLASSES.append((_wi[0], _wi[1], _s, len(_tiles)))
_T = len(_tiles)
_TIB = np.array([t[0] for t in _tiles], dtype=np.int32)
_TOB = np.array([t[1] for t in _tiles], dtype=np.int32)
_TDJ = np.array([t[2] for t in _tiles], dtype=np.int32)
_TDI = np.array([t[3] for t in _tiles], dtype=np.int32)
_TN = np.array([t[4] for t in _tiles], dtype=np.int32)


_LOG2E = float(np.log2(np.e))
_LN2 = float(np.log(2.0))
_SPCLAMP = 80.0 * _LOG2E


def _sig2(a2):
    return 1.0 / (1.0 + jnp.exp2(a2))


def _sp2(b2):
    return jnp.log2(1.0 + jnp.exp2(jnp.minimum(b2, _SPCLAMP)))


def _body(x_ref, c_ref, wc1_ref, we1_ref, b1_ref, gb1_ref,
          wc2_ref, we2_ref, b2_ref, gb2_ref,
          tib_ref, tob_ref, tdj_ref, tdi_ref, tn_ref,
          out_ref, pq_ref, agg_ref, x2_ref):

    def project(xv, wc_ref, we_ref, b_ref):
        cc = jnp.dot(c_ref[...], we_ref[...],
                     preferred_element_type=jnp.float32)
        cf = cc[:, 0:64]
        cs = cc[:, 64:128]
        adj = jnp.concatenate([cf, -cf, cs, -cs], axis=1) + b_ref[...]
        pq_ref[...] = jnp.dot(xv, wc_ref[...],
                              preferred_element_type=jnp.float32) + adj

    def pair_loop():
        def make_tb(jh, ih):
            def tb(t, carry):
                ib = pl.multiple_of(tib_ref[t], 8)
                ob = pl.multiple_of(tob_ref[t], 8)
                dj = tdj_ref[t]
                di = tdi_ref[t]
                n = tn_ref[t]
                pf = pq_ref[pl.ds(ib, ih), 0:64]
                ps = pq_ref[pl.ds(ib, ih), 128:192]
                jf = jnp.concatenate(
                    [pq_ref[pl.ds(ob, jh), 64:128],
                     pq_ref[pl.ds(ob + jh, jh), 64:128]], axis=1)
                js = jnp.concatenate(
                    [pq_ref[pl.ds(ob, jh), 192:256],
                     pq_ref[pl.ds(ob + jh, jh), 192:256]], axis=1)
                pfd = jnp.concatenate([pf, pf], axis=1)[:, None, :]
                psd = jnp.concatenate([ps, ps], axis=1)[:, None, :]
                a = pfd + jf[None]
                b = psd + js[None]
                m = _sig2(a) * _sp2(b)
                r = jax.lax.broadcasted_iota(jnp.int32, (jh, 128), 0)
                l = jax.lax.broadcasted_iota(jnp.int32, (jh, 128), 1)
                jloc = r + jnp.where(l >= 64, jh, 0)
                jok = (jloc >= dj) & (jloc < dj + n)
                m = jnp.where(jok[None], m, 0.0)
                s = jnp.sum(m, axis=1)
                aggt = (s[:, 0:64] + s[:, 64:128]) * _LN2
                rowsi = jax.lax.broadcasted_iota(jnp.int32, (ih, 1), 0)
                iok = (rowsi >= di) & (rowsi < di + n)
                old = agg_ref[pl.ds(ib, ih), :]
                agg_ref[pl.ds(ib, ih), :] = jnp.where(iok, aggt, old)
                return carry
            return tb
        for w, ih, s, e in _CLASSES:
            jax.lax.fori_loop(s, e, make_tb(w // 2, ih), 0)

    def bn_res_relu(xv, gb_ref):
        pq = pq_ref[...]
        diag = _sig2(pq[:, 0:64] + pq[:, 64:128]) * \
            _sp2(pq[:, 128:192] + pq[:, 192:256]) * _LN2
        agg = agg_ref[...] - diag
        rows = jax.lax.broadcasted_iota(jnp.int32, (_NPAD, 1), 0)
        valid = rows < _N
        aggm = jnp.where(valid, agg, 0.0)
        mu = jnp.sum(aggm, axis=0, keepdims=True) * (1.0 / _N)
        d = jnp.where(valid, aggm - mu, 0.0)
        var = jnp.sum(d * d, axis=0, keepdims=True) * (1.0 / _N)
        g = gb_ref[0:1, :]
        be = gb_ref[1:2, :]
        scale = g * jax.lax.rsqrt(var + 1e-5)
        return jnp.maximum(xv + (agg - mu) * scale + be, 0.0)

    x1 = x_ref[...]
    project(x1, wc1_ref, we1_ref, b1_ref)
    pair_loop()
    x2 = bn_res_relu(x1, gb1_ref)
    x2_ref[...] = x2
    project(x2_ref[...], wc2_ref, we2_ref, b2_ref)
    pair_loop()
    out_ref[...] = bn_res_relu(x2_ref[...], gb2_ref)


def kernel(gnn_in, centers, Wf1, bf1, Ws1, bs1, g1, be1,
           Wf2, bf2, Ws2, bs2, g2, be2, edge_index):
    f32 = jnp.float32
    xpad = jnp.zeros((_NPAD, _F), f32).at[:_N].set(gnn_in)
    cpad = jnp.zeros((_NPAD, 2), f32).at[:_N].set(centers)

    def pack(Wf, bf, Ws, bs, g, be):
        Wfs = Wf * (-_LOG2E)
        Wss = Ws * _LOG2E
        wc = jnp.concatenate([Wfs[0:64], Wfs[64:128], Wss[0:64],
                              Wss[64:128]], axis=1)
        we = jnp.concatenate([Wfs[128:130], Wss[128:130]], axis=1)
        z = jnp.zeros((_F,), f32)
        b = jnp.concatenate([bf * (-_LOG2E), z, bs * _LOG2E, z])[None, :]
        gb = jnp.stack([g, be])
        return wc, we, b, gb

    wc1, we1, b1, gb1 = pack(Wf1, bf1, Ws1, bs1, g1, be1)
    wc2, we2, b2, gb2 = pack(Wf2, bf2, Ws2, bs2, g2, be2)

    vmem = pl.BlockSpec(memory_space=pltpu.VMEM)
    sspec = pl.BlockSpec(memory_space=pltpu.SMEM)

    out = pl.pallas_call(
        _body,
        out_shape=jax.ShapeDtypeStruct((_NPAD, _F), f32),
        in_specs=[vmem] * 10 + [sspec] * 5,
        out_specs=vmem,
        scratch_shapes=[
            pltpu.VMEM((_NPAD, 4 * _F), f32),
            pltpu.VMEM((_NPAD, _F), f32),
            pltpu.VMEM((_NPAD, _F), f32),
        ],
    )(xpad, cpad, wc1, we1, b1, gb1, wc2, we2, b2, gb2,
      jnp.asarray(_TIB), jnp.asarray(_TOB), jnp.asarray(_TDJ),
      jnp.asarray(_TDI), jnp.asarray(_TN))
    return out[:_N]
